# baseline (speedup 1.0000x reference)
import functools
import math

import jax
import jax.numpy as jnp
import numpy as np
from jax import lax
from jax.experimental import pallas as pl
from jax.experimental.pallas import tpu as pltpu
from jax.experimental.pallas import tpu_sc as plsc

_N = 96
_F = 32
_K = 32
_CUTOFF = 15.0
_NP = _N * _N
_E = _N * (_N - 1)
_NSUB = 32
_APS = _N // _NSUB
_JB = _N // 16
_PR = _NP * _K // 128
_RPB = _N * _K // 128

_LOGBINOM = np.asarray(
    [
        math.lgamma(float(_K)) - math.lgamma(k + 1.0) - math.lgamma(float(_K) - k)
        for k in range(_K)
    ],
    dtype=np.float32,
)
_BINOM = [float(np.float32(np.exp(_LOGBINOM[k]))) for k in range(_K)]

_LN_CLIP_LO = float(np.log(1e-10))

_PPERM = np.zeros((_N, _N), np.float32)
for _jp in range(_N):
    _PPERM[_jp, 4 * (_jp % _RPB) + _jp // _RPB] = 1.0

_RSEL = np.zeros((_N, _PR), np.float32)
for _i in range(_N):
    _RSEL[_i, _i * _RPB:(_i + 1) * _RPB] = 1.0


def _pow_static(base_powers, n):
    acc = None
    for b in range(5):
        if n & (1 << b):
            acc = base_powers[b] if acc is None else acc * base_powers[b]
    return acc


def _sc_body(r_hbm, al_hbm, basis_hbm, r_v, al_v, buf0_v, buf1_v, sem0, sem1):
    i32 = jnp.int32
    f32 = jnp.float32
    wid = lax.axis_index("c") * 16 + lax.axis_index("s")

    pltpu.sync_copy(r_hbm, r_v)
    pltpu.sync_copy(al_hbm, al_v)

    alpha = plsc.load_gather(al_v, [jnp.zeros((16,), i32)])

    c = 1.0 + jnp.exp(alpha)
    cb = plsc.bitcast(c, i32)
    e2 = ((lax.shift_right_logical(cb, 23) & 255) - 127).astype(f32)
    mant = plsc.bitcast((cb & 0x7FFFFF) | 0x3F800000, f32)
    y = 0.6931472 * e2 + 0.6931472 * (mant - 1.0)
    for _ in range(4):
        y = y - 1.0 + c * jnp.exp(-y)
    sa = jnp.where(alpha > 20.0, alpha, y)

    jiota = lax.broadcasted_iota(i32, (16,), 0)
    rowv = lax.shift_right_logical(jiota * _K, 7)
    colbase = (jiota * _K) & 127
    bufsems = ((buf0_v, sem0), (buf1_v, sem1))

    zeros16 = jnp.zeros((16,), jnp.int32)

    def _basis_tile(iv, rix, riy, riz, j0, buf):
        jv = jiota + j0
        dx = plsc.load_gather(r_v, [jv, zeros16]) - rix
        dy = plsc.load_gather(r_v, [jv, zeros16 + 1]) - riy
        dz = plsc.load_gather(r_v, [jv, zeros16 + 2]) - riz
        d2 = dx * dx + dy * dy + dz * dz + 1e-12
        ib = 0x5F3759DF - lax.shift_right_logical(plsc.bitcast(d2, i32), 1)
        r = plsc.bitcast(ib, f32)
        for _ in range(3):
            r = r * (1.5 - 0.5 * d2 * r * r)
        d = d2 * r

        lex = jnp.maximum(-sa * d, _LN_CLIP_LO)
        ex = jnp.exp(lex)
        q = 1.0 - ex
        dd = d * d
        fin = (d < _CUTOFF) & (jv != iv)
        fc = jnp.where(fin, jnp.exp(-dd / (_CUTOFF * _CUTOFF - dd + 1e-9)), 0.0)

        ep = [ex]
        qp = [q]
        for _ in range(4):
            ep.append(ep[-1] * ep[-1])
            qp.append(qp[-1] * qp[-1])

        for k in range(_K):
            acc = fc * _BINOM[k]
            pe = _pow_static(ep, k)
            if pe is not None:
                acc = acc * pe
            pq = _pow_static(qp, _K - 1 - k)
            if pq is not None:
                acc = acc * pq
            plsc.store_scatter(buf, [rowv, colbase + k], acc)

    for a in range(_APS):
        i = wid * _APS + a
        iv = jnp.broadcast_to(i, (16,))
        rix = plsc.load_gather(r_v, [iv, zeros16])
        riy = plsc.load_gather(r_v, [iv, zeros16 + 1])
        riz = plsc.load_gather(r_v, [iv, zeros16 + 2])

        def hblk_body(h, _, _iv=iv, _rix=rix, _riy=riy, _riz=riz, _a=a, _i=i):
            for t, (buf, sem) in enumerate(bufsems):
                j0 = 32 * h + 16 * t
                if _a == 0:
                    @pl.when(h > 0)
                    def _wait_prev():
                        pltpu.make_async_copy(
                            buf, basis_hbm.at[pl.ds(0, 4)], sem
                        ).wait()
                else:
                    pltpu.make_async_copy(
                        buf, basis_hbm.at[pl.ds(0, 4)], sem
                    ).wait()
                _basis_tile(_iv, _rix, _riy, _riz, j0, buf)
                pltpu.async_copy(
                    buf, basis_hbm.at[pl.ds(_i * _RPB + 8 * h + 4 * t, 4)], sem
                )
            return 0

        lax.fori_loop(0, _JB // 2, hblk_body, 0)

    pltpu.make_async_copy(buf0_v, basis_hbm.at[pl.ds(0, 4)], sem0).wait()
    pltpu.make_async_copy(buf1_v, basis_hbm.at[pl.ds(0, 4)], sem1).wait()


def _blockdiag4(W):
    Wrow = jnp.concatenate([W, W, W, W], axis=1)
    Wbig = jnp.concatenate([Wrow, Wrow, Wrow, Wrow], axis=0)
    a = lax.broadcasted_iota(jnp.int32, (128, 128), 0)
    b = lax.broadcasted_iota(jnp.int32, (128, 128), 1)
    return jnp.where((a // 32) == (b // 32), Wbig, 0.0)


def _tc_body(basis_ref, Z_ref, emb_ref, Wrii_ref, Wrij_ref, W1_ref, W2_ref,
             Wp_ref, Pperm_ref, out_ref):
    f32 = jnp.float32
    basis2 = basis_ref[...]

    g_ii2 = jnp.dot(basis2, _blockdiag4(Wrii_ref[...]), preferred_element_type=f32)
    g_ij2 = jnp.dot(basis2, _blockdiag4(Wrij_ref[...]), preferred_element_type=f32)

    nz = emb_ref.shape[0]
    iotav = lax.broadcasted_iota(jnp.int32, (nz, _N), 0)
    onehotT = (jnp.broadcast_to(Z_ref[...][None, :], (nz, _N)) == iotav).astype(f32)
    x0 = lax.dot_general(
        onehotT, emb_ref[...], (((0,), (0,)), ((), ())),
        preferred_element_type=f32,
    )

    Pperm = Pperm_ref[...]
    ri = lax.broadcasted_iota(jnp.int32, (_N, _PR), 0)
    rr = lax.broadcasted_iota(jnp.int32, (_N, _PR), 1)
    Rsel = ((rr >= ri * _RPB) & (rr < (ri + 1) * _RPB)).astype(f32)

    def seg_reduce(x, g2):
        xall = jnp.dot(Pperm, x, preferred_element_type=f32)
        xperm = jnp.concatenate(
            [xall[0:24], xall[24:48], xall[48:72], xall[72:96]], axis=1
        )
        xt = jnp.broadcast_to(xperm[None], (_N, _RPB, 128)).reshape(_PR, 128)
        A1 = jnp.dot(Rsel, g2 * xt, preferred_element_type=f32)
        return (A1[:, 0:32] + A1[:, 32:64]) + (A1[:, 64:96] + A1[:, 96:128])

    x1 = x0 + seg_reduce(x0, g_ij2)

    t = jnp.dot(x1, W1_ref[...], preferred_element_type=f32)
    sw = t / (1.0 + jnp.exp(-t))
    x2 = x1 + jnp.dot(sw, W2_ref[...], preferred_element_type=f32)

    out_ref[...] = jnp.dot(
        x2 * seg_reduce(x2, g_ii2), Wp_ref[...], preferred_element_type=f32
    )


@jax.jit
def kernel(R, Z, emb, alpha, W_rii, W_rij, W1, W2, Wp):
    f32 = jnp.float32
    Rc = R.astype(f32)
    al = jnp.asarray(alpha, f32).reshape(1)
    Zc = Z.astype(jnp.int32)

    mesh = plsc.VectorSubcoreMesh(
        core_axis_name="c", subcore_axis_name="s", num_cores=2, num_subcores=16
    )
    sc = functools.partial(
        pl.kernel,
        out_type=jax.ShapeDtypeStruct((_PR, 128), f32),
        mesh=mesh,
        compiler_params=pltpu.CompilerParams(
            needs_layout_passes=False, use_tc_tiling_on_sc=False
        ),
        scratch_types=[
            pltpu.VMEM((_N, 3), f32),
            pltpu.VMEM((1,), f32),
            pltpu.VMEM((4, 128), f32),
            pltpu.VMEM((4, 128), f32),
            pltpu.SemaphoreType.DMA,
            pltpu.SemaphoreType.DMA,
        ],
    )(_sc_body)
    basis2 = sc(Rc, al)

    rows = pl.pallas_call(
        _tc_body,
        out_shape=jax.ShapeDtypeStruct((_N, _F), f32),
    )(basis2, Zc, emb, W_rii, W_rij, W1, W2, Wp, jnp.asarray(_PPERM))
    return jnp.broadcast_to(rows[:, None, :], (_N, _N - 1, _F)).reshape(_E, _F)


# speedup vs baseline: 1.0112x; 1.0112x over previous
import functools
import math

import jax
import jax.numpy as jnp
import numpy as np
from jax import lax
from jax.experimental import pallas as pl
from jax.experimental.pallas import tpu as pltpu
from jax.experimental.pallas import tpu_sc as plsc

_N = 96
_F = 32
_K = 32
_CUTOFF = 15.0
_NP = _N * _N
_E = _N * (_N - 1)
_NSUB = 32
_APS = _N // _NSUB
_JB = _N // 16
_PR = _NP * _K // 128
_RPB = _N * _K // 128

_LOGBINOM = np.asarray(
    [
        math.lgamma(float(_K)) - math.lgamma(k + 1.0) - math.lgamma(float(_K) - k)
        for k in range(_K)
    ],
    dtype=np.float32,
)
_BINOM = [float(np.float32(np.exp(_LOGBINOM[k]))) for k in range(_K)]

_LN_CLIP_LO = float(np.log(1e-10))

_PPERM = np.zeros((_N, _N), np.float32)
for _jp in range(_N):
    _PPERM[_jp, 4 * (_jp % _RPB) + _jp // _RPB] = 1.0

_RSEL = np.zeros((_N, _PR), np.float32)
for _i in range(_N):
    _RSEL[_i, _i * _RPB:(_i + 1) * _RPB] = 1.0


def _pow_static(base_powers, n):
    acc = None
    for b in range(5):
        if n & (1 << b):
            acc = base_powers[b] if acc is None else acc * base_powers[b]
    return acc


def _sc_body(aux_hbm, basis_hbm, aux_v, buf0_v, buf1_v, sem0, sem1):
    i32 = jnp.int32
    f32 = jnp.float32
    wid = lax.axis_index("c") * 16 + lax.axis_index("s")

    pltpu.sync_copy(aux_hbm, aux_v)

    alpha = plsc.load_gather(aux_v, [jnp.full((16,), 3 * _N, i32)])

    c = 1.0 + jnp.exp(alpha)
    cb = plsc.bitcast(c, i32)
    e2 = ((lax.shift_right_logical(cb, 23) & 255) - 127).astype(f32)
    mant = plsc.bitcast((cb & 0x7FFFFF) | 0x3F800000, f32)
    y = 0.6931472 * e2 + 0.6931472 * (mant - 1.0)
    for _ in range(4):
        y = y - 1.0 + c * jnp.exp(-y)
    sa = jnp.where(alpha > 20.0, alpha, y)

    jiota = lax.broadcasted_iota(i32, (16,), 0)
    rowv = lax.shift_right_logical(jiota * _K, 7)
    colbase = (jiota * _K) & 127
    bufsems = ((buf0_v, sem0), (buf1_v, sem1))

    def _basis_tile(iv, rix, riy, riz, j0, buf):
        jv = jiota + j0
        jv3 = jv * 3
        dx = plsc.load_gather(aux_v, [jv3]) - rix
        dy = plsc.load_gather(aux_v, [jv3 + 1]) - riy
        dz = plsc.load_gather(aux_v, [jv3 + 2]) - riz
        d2 = dx * dx + dy * dy + dz * dz + 1e-12
        ib = 0x5F3759DF - lax.shift_right_logical(plsc.bitcast(d2, i32), 1)
        r = plsc.bitcast(ib, f32)
        for _ in range(3):
            r = r * (1.5 - 0.5 * d2 * r * r)
        d = d2 * r

        lex = jnp.maximum(-sa * d, _LN_CLIP_LO)
        ex = jnp.exp(lex)
        q = 1.0 - ex
        dd = d * d
        fin = (d < _CUTOFF) & (jv != iv)
        fc = jnp.where(fin, jnp.exp(-dd / (_CUTOFF * _CUTOFF - dd + 1e-9)), 0.0)

        ep = [ex]
        qp = [q]
        for _ in range(4):
            ep.append(ep[-1] * ep[-1])
            qp.append(qp[-1] * qp[-1])

        for k in range(_K):
            acc = fc * _BINOM[k]
            pe = _pow_static(ep, k)
            if pe is not None:
                acc = acc * pe
            pq = _pow_static(qp, _K - 1 - k)
            if pq is not None:
                acc = acc * pq
            plsc.store_scatter(buf, [rowv, colbase + k], acc)

    for a in range(_APS):
        i = wid * _APS + a
        i3 = jnp.broadcast_to(i * 3, (16,))
        iv = jnp.broadcast_to(i, (16,))
        rix = plsc.load_gather(aux_v, [i3])
        riy = plsc.load_gather(aux_v, [i3 + 1])
        riz = plsc.load_gather(aux_v, [i3 + 2])

        def hblk_body(h, _, _iv=iv, _rix=rix, _riy=riy, _riz=riz, _a=a, _i=i):
            for t, (buf, sem) in enumerate(bufsems):
                j0 = 32 * h + 16 * t
                if _a == 0:
                    @pl.when(h > 0)
                    def _wait_prev():
                        pltpu.make_async_copy(
                            buf, basis_hbm.at[pl.ds(0, 4)], sem
                        ).wait()
                else:
                    pltpu.make_async_copy(
                        buf, basis_hbm.at[pl.ds(0, 4)], sem
                    ).wait()
                _basis_tile(_iv, _rix, _riy, _riz, j0, buf)
                pltpu.async_copy(
                    buf, basis_hbm.at[pl.ds(_i * _RPB + 8 * h + 4 * t, 4)], sem
                )
            return 0

        lax.fori_loop(0, _JB // 2, hblk_body, 0)

    pltpu.make_async_copy(buf0_v, basis_hbm.at[pl.ds(0, 4)], sem0).wait()
    pltpu.make_async_copy(buf1_v, basis_hbm.at[pl.ds(0, 4)], sem1).wait()


def _blockdiag4(W):
    Wrow = jnp.concatenate([W, W, W, W], axis=1)
    Wbig = jnp.concatenate([Wrow, Wrow, Wrow, Wrow], axis=0)
    a = lax.broadcasted_iota(jnp.int32, (128, 128), 0)
    b = lax.broadcasted_iota(jnp.int32, (128, 128), 1)
    return jnp.where((a // 32) == (b // 32), Wbig, 0.0)


def _tc_body(basis_ref, Z_ref, emb_ref, Wrii_ref, Wrij_ref, W1_ref, W2_ref,
             Wp_ref, Pperm_ref, out_ref):
    f32 = jnp.float32
    basis2 = basis_ref[...]

    g_ii2 = jnp.dot(basis2, _blockdiag4(Wrii_ref[...]), preferred_element_type=f32)
    g_ij2 = jnp.dot(basis2, _blockdiag4(Wrij_ref[...]), preferred_element_type=f32)

    nz = emb_ref.shape[0]
    iotav = lax.broadcasted_iota(jnp.int32, (nz, _N), 0)
    onehotT = (jnp.broadcast_to(Z_ref[...][None, :], (nz, _N)) == iotav).astype(f32)
    x0 = lax.dot_general(
        onehotT, emb_ref[...], (((0,), (0,)), ((), ())),
        preferred_element_type=f32,
    )

    Pperm = Pperm_ref[...]
    ri = lax.broadcasted_iota(jnp.int32, (_N, _PR), 0)
    rj = lax.broadcasted_iota(jnp.int32, (_N, _PR), 1)
    Rsel = ((rj >= ri * _RPB) & (rj < (ri + 1) * _RPB)).astype(f32)

    def seg_reduce(x, g2):
        xall = jnp.dot(Pperm, x, preferred_element_type=f32)
        xperm = jnp.concatenate(
            [xall[0:24], xall[24:48], xall[48:72], xall[72:96]], axis=1
        )
        xt = jnp.broadcast_to(xperm[None], (_N, _RPB, 128)).reshape(_PR, 128)
        A1 = jnp.dot(Rsel, g2 * xt, preferred_element_type=f32)
        return (A1[:, 0:32] + A1[:, 32:64]) + (A1[:, 64:96] + A1[:, 96:128])

    x1 = x0 + seg_reduce(x0, g_ij2)

    t = jnp.dot(x1, W1_ref[...], preferred_element_type=f32)
    sw = t / (1.0 + jnp.exp(-t))
    x2 = x1 + jnp.dot(sw, W2_ref[...], preferred_element_type=f32)

    out_ref[...] = jnp.dot(
        x2 * seg_reduce(x2, g_ii2), Wp_ref[...], preferred_element_type=f32
    )


@jax.jit
def kernel(R, Z, emb, alpha, W_rii, W_rij, W1, W2, Wp):
    f32 = jnp.float32
    aux = jnp.concatenate([
        jnp.ravel(R).astype(f32),
        jnp.asarray(alpha, f32).reshape(1),
        jnp.zeros((15,), f32),
    ])
    Zc = Z.astype(jnp.int32)

    mesh = plsc.VectorSubcoreMesh(
        core_axis_name="c", subcore_axis_name="s", num_cores=2, num_subcores=16
    )
    sc = functools.partial(
        pl.kernel,
        out_type=jax.ShapeDtypeStruct((_PR, 128), f32),
        mesh=mesh,
        compiler_params=pltpu.CompilerParams(
            needs_layout_passes=False, use_tc_tiling_on_sc=False
        ),
        scratch_types=[
            pltpu.VMEM((3 * _N + 16,), f32),
            pltpu.VMEM((4, 128), f32),
            pltpu.VMEM((4, 128), f32),
            pltpu.SemaphoreType.DMA,
            pltpu.SemaphoreType.DMA,
        ],
    )(_sc_body)
    basis2 = sc(aux)

    rows = pl.pallas_call(
        _tc_body,
        out_shape=jax.ShapeDtypeStruct((_N, _F), f32),
    )(basis2, Zc, emb, W_rii, W_rij, W1, W2, Wp, jnp.asarray(_PPERM))
    return jnp.broadcast_to(rows[:, None, :], (_N, _N - 1, _F)).reshape(_E, _F)


# speedup vs baseline: 1.0175x; 1.0062x over previous
import functools
import math

import jax
import jax.numpy as jnp
import numpy as np
from jax import lax
from jax.experimental import pallas as pl
from jax.experimental.pallas import tpu as pltpu
from jax.experimental.pallas import tpu_sc as plsc

_N = 96
_F = 32
_K = 32
_CUTOFF = 15.0
_NP = _N * _N
_E = _N * (_N - 1)
_NSUB = 32
_APS = _N // _NSUB
_JB = _N // 16
_PR = _NP * _K // 128
_RPB = _N * _K // 128

_LOGBINOM = np.asarray(
    [
        math.lgamma(float(_K)) - math.lgamma(k + 1.0) - math.lgamma(float(_K) - k)
        for k in range(_K)
    ],
    dtype=np.float32,
)
_BINOM = [float(np.float32(np.exp(_LOGBINOM[k]))) for k in range(_K)]

_LN_CLIP_LO = float(np.log(1e-10))

_PPERM = np.zeros((_N, _N), np.float32)
for _jp in range(_N):
    _PPERM[_jp, 4 * (_jp % _RPB) + _jp // _RPB] = 1.0

_RSEL = np.zeros((_N, _PR), np.float32)
for _i in range(_N):
    _RSEL[_i, _i * _RPB:(_i + 1) * _RPB] = 1.0


def _pow_static(base_powers, n):
    acc = None
    for b in range(5):
        if n & (1 << b):
            acc = base_powers[b] if acc is None else acc * base_powers[b]
    return acc


def _sc_body(aux_hbm, basis_hbm, aux_v, buf0_v, buf1_v, sem0, sem1):
    i32 = jnp.int32
    f32 = jnp.float32
    wid = lax.axis_index("c") * 16 + lax.axis_index("s")

    pltpu.sync_copy(aux_hbm, aux_v)

    alpha = plsc.load_gather(aux_v, [jnp.full((16,), 3 * _N, i32)])

    c = 1.0 + jnp.exp(alpha)
    cb = plsc.bitcast(c, i32)
    e2 = ((lax.shift_right_logical(cb, 23) & 255) - 127).astype(f32)
    mant = plsc.bitcast((cb & 0x7FFFFF) | 0x3F800000, f32)
    y = 0.6931472 * e2 + 0.6931472 * (mant - 1.0)
    for _ in range(4):
        y = y - 1.0 + c * jnp.exp(-y)
    sa = jnp.where(alpha > 20.0, alpha, y)

    jiota = lax.broadcasted_iota(i32, (16,), 0)
    rowv = lax.shift_right_logical(jiota * _K, 7)
    colbase = (jiota * _K) & 127
    bufsems = ((buf0_v, sem0), (buf1_v, sem1))

    def _basis_tile(iv, rix, riy, riz, j0, buf):
        jv = jiota + j0
        jv3 = jv * 3
        dx = plsc.load_gather(aux_v, [jv3]) - rix
        dy = plsc.load_gather(aux_v, [jv3 + 1]) - riy
        dz = plsc.load_gather(aux_v, [jv3 + 2]) - riz
        d2 = dx * dx + dy * dy + dz * dz + 1e-12
        ib = 0x5F3759DF - lax.shift_right_logical(plsc.bitcast(d2, i32), 1)
        r = plsc.bitcast(ib, f32)
        for _ in range(3):
            r = r * (1.5 - 0.5 * d2 * r * r)
        d = d2 * r

        lex = jnp.maximum(-sa * d, _LN_CLIP_LO)
        ex = jnp.exp(lex)
        q = 1.0 - ex
        dd = d * d
        fin = (d < _CUTOFF) & (jv != iv)
        fc = jnp.where(fin, jnp.exp(-dd / (_CUTOFF * _CUTOFF - dd + 1e-9)), 0.0)

        ep = [ex]
        qp = [q]
        for _ in range(4):
            ep.append(ep[-1] * ep[-1])
            qp.append(qp[-1] * qp[-1])

        for k in range(_K):
            acc = fc * _BINOM[k]
            pe = _pow_static(ep, k)
            if pe is not None:
                acc = acc * pe
            pq = _pow_static(qp, _K - 1 - k)
            if pq is not None:
                acc = acc * pq
            plsc.store_scatter(buf, [rowv, colbase + k], acc)

    for a in range(_APS):
        i = wid * _APS + a
        i3 = jnp.broadcast_to(i * 3, (16,))
        iv = jnp.broadcast_to(i, (16,))
        rix = plsc.load_gather(aux_v, [i3])
        riy = plsc.load_gather(aux_v, [i3 + 1])
        riz = plsc.load_gather(aux_v, [i3 + 2])

        def hblk_body(h, _, _iv=iv, _rix=rix, _riy=riy, _riz=riz, _a=a, _i=i):
            for t, (buf, sem) in enumerate(bufsems):
                j0 = 32 * h + 16 * t
                if _a == 0:
                    @pl.when(h > 0)
                    def _wait_prev():
                        pltpu.make_async_copy(
                            buf, basis_hbm.at[pl.ds(0, 4)], sem
                        ).wait()
                else:
                    pltpu.make_async_copy(
                        buf, basis_hbm.at[pl.ds(0, 4)], sem
                    ).wait()
                _basis_tile(_iv, _rix, _riy, _riz, j0, buf)
                pltpu.async_copy(
                    buf, basis_hbm.at[pl.ds(_i * _RPB + 8 * h + 4 * t, 4)], sem
                )
            return 0

        lax.fori_loop(0, _JB // 2, hblk_body, 0)

    pltpu.make_async_copy(buf0_v, basis_hbm.at[pl.ds(0, 4)], sem0).wait()
    pltpu.make_async_copy(buf1_v, basis_hbm.at[pl.ds(0, 4)], sem1).wait()


def _blockdiag4(W):
    Wrow = jnp.concatenate([W, W, W, W], axis=1)
    Wbig = jnp.concatenate([Wrow, Wrow, Wrow, Wrow], axis=0)
    a = lax.broadcasted_iota(jnp.int32, (128, 128), 0)
    b = lax.broadcasted_iota(jnp.int32, (128, 128), 1)
    return jnp.where((a // 32) == (b // 32), Wbig, 0.0)


def _tc_body(basis_ref, Z_ref, emb_ref, Wrii_ref, Wrij_ref, W1_ref, W2_ref,
             Wp_ref, Pperm_ref, out_ref):
    f32 = jnp.float32
    basis2 = basis_ref[...]

    W4 = jnp.concatenate(
        [_blockdiag4(Wrii_ref[...]), _blockdiag4(Wrij_ref[...])], axis=1
    )
    g2cat = jnp.dot(basis2, W4, preferred_element_type=f32)
    g_ii2 = g2cat[:, 0:128]
    g_ij2 = g2cat[:, 128:256]

    nz = emb_ref.shape[0]
    iotav = lax.broadcasted_iota(jnp.int32, (nz, _N), 0)
    onehotT = (jnp.broadcast_to(Z_ref[...][None, :], (nz, _N)) == iotav).astype(f32)
    x0 = lax.dot_general(
        onehotT, emb_ref[...], (((0,), (0,)), ((), ())),
        preferred_element_type=f32,
    )

    Pperm = Pperm_ref[...]
    ri = lax.broadcasted_iota(jnp.int32, (_N, _PR), 0)
    rj = lax.broadcasted_iota(jnp.int32, (_N, _PR), 1)
    Rsel = ((rj >= ri * _RPB) & (rj < (ri + 1) * _RPB)).astype(f32)

    def seg_reduce(x, g2):
        xall = jnp.dot(Pperm, x, preferred_element_type=f32)
        xperm = jnp.concatenate(
            [xall[0:24], xall[24:48], xall[48:72], xall[72:96]], axis=1
        )
        xt = jnp.broadcast_to(xperm[None], (_N, _RPB, 128)).reshape(_PR, 128)
        A1 = jnp.dot(Rsel, g2 * xt, preferred_element_type=f32)
        return (A1[:, 0:32] + A1[:, 32:64]) + (A1[:, 64:96] + A1[:, 96:128])

    x1 = x0 + seg_reduce(x0, g_ij2)

    t = jnp.dot(x1, W1_ref[...], preferred_element_type=f32)
    sw = t / (1.0 + jnp.exp(-t))
    x2 = x1 + jnp.dot(sw, W2_ref[...], preferred_element_type=f32)

    out_ref[...] = jnp.dot(
        x2 * seg_reduce(x2, g_ii2), Wp_ref[...], preferred_element_type=f32
    )


@jax.jit
def kernel(R, Z, emb, alpha, W_rii, W_rij, W1, W2, Wp):
    f32 = jnp.float32
    aux = jnp.concatenate([
        jnp.ravel(R).astype(f32),
        jnp.asarray(alpha, f32).reshape(1),
        jnp.zeros((15,), f32),
    ])
    Zc = Z.astype(jnp.int32)

    mesh = plsc.VectorSubcoreMesh(
        core_axis_name="c", subcore_axis_name="s", num_cores=2, num_subcores=16
    )
    sc = functools.partial(
        pl.kernel,
        out_type=jax.ShapeDtypeStruct((_PR, 128), f32),
        mesh=mesh,
        compiler_params=pltpu.CompilerParams(
            needs_layout_passes=False, use_tc_tiling_on_sc=False
        ),
        scratch_types=[
            pltpu.VMEM((3 * _N + 16,), f32),
            pltpu.VMEM((4, 128), f32),
            pltpu.VMEM((4, 128), f32),
            pltpu.SemaphoreType.DMA,
            pltpu.SemaphoreType.DMA,
        ],
    )(_sc_body)
    basis2 = sc(aux)

    rows = pl.pallas_call(
        _tc_body,
        out_shape=jax.ShapeDtypeStruct((_N, _F), f32),
    )(basis2, Zc, emb, W_rii, W_rij, W1, W2, Wp, jnp.asarray(_PPERM))
    return jnp.broadcast_to(rows[:, None, :], (_N, _N - 1, _F)).reshape(_E, _F)
